# baseline (speedup 1.0000x reference)
import dataclasses
import functools

import jax
import jax.numpy as jnp
from jax import lax
from jax.experimental import pallas as pl
from jax.experimental.pallas import tpu as pltpu
from jax.experimental.pallas import tpu_sc as plsc

_N, _E, _D, _G = 10000, 320000, 128, 64
_NC, _NS = 2, 16
_NW = _NC * _NS
_CHUNK = 128
_NCHUNK = 80
_EPW = _CHUNK * _NCHUNK
_EPAD = _NW * _EPW
_NPAD = 10240
_RPS = _NPAD // _NS
_DUMP = _N
_NBUF = 2
_WSTEP = 8
_NCH0 = 160
_NCH1 = 0
_NPA = 10112
_RPA = _NPA // _NS

_F32 = jnp.float32


def _vmesh():
    return plsc.VectorSubcoreMesh(core_axis_name="c", subcore_axis_name="s")


def _sc_compiler_params():
    cp = pltpu.CompilerParams()
    if "needs_layout_passes" in pltpu.CompilerParams.__dataclass_fields__:
        cp = dataclasses.replace(cp, needs_layout_passes=False)
    return cp


def _sc_degree(dstc, zdeg):
    @functools.partial(
        pl.kernel,
        out_type=jax.ShapeDtypeStruct((_NW * _NPAD,), _F32),
        mesh=_vmesh(),
        scratch_types=[
            pltpu.VMEM((_NCHUNK, _CHUNK), jnp.int32),
            pltpu.VMEM((_NPAD,), _F32),
        ],
        compiler_params=_sc_compiler_params(),
    )
    def deg_kernel(dst_hbm, z_hbm, out_hbm, dst_v, local):
        c = lax.axis_index("c")
        s = lax.axis_index("s")
        wid = s * _NC + c
        pltpu.sync_copy(dst_hbm.at[pl.ds(wid * _NCHUNK, _NCHUNK)], dst_v)
        pltpu.sync_copy(z_hbm, local)
        ones = jnp.full((16,), 1.0, _F32)

        @pl.loop(0, _NCHUNK)
        def _(j):
            @pl.loop(0, _CHUNK // 16)
            def _(k):
                idx16 = dst_v[j, pl.ds(k * 16, 16)]
                plsc.addupdate_scatter(local, [idx16], ones)

        pltpu.sync_copy(local, out_hbm.at[pl.ds(wid * _NPAD, _NPAD)])

    return deg_kernel(dstc, zdeg)


def _sc_edge_agg(hn, srcf, dstc, zrows):
    nbuf = _NBUF

    @functools.partial(
        pl.kernel,
        out_type=jax.ShapeDtypeStruct((_NC * _NPA, _D), _F32),
        mesh=_vmesh(),
        scratch_types=[
            pltpu.VMEM((_WSTEP * _CHUNK,), jnp.int32),
            pltpu.VMEM((_WSTEP, _CHUNK), jnp.int32),
            pltpu.VMEM((nbuf, _CHUNK, _D), _F32),
            pltpu.VMEM_SHARED((_NPA, _D), _F32),
            pltpu.SemaphoreType.DMA((nbuf,)),
            pltpu.SemaphoreType.DMA((nbuf,)),
        ],
    )
    def agg_kernel(hn_hbm, src_hbm, dst_hbm, z_hbm, out_hbm,
                   src_v, dst_v, rows_v, acc, gsem, ssem):
        c = lax.axis_index("c")
        s = lax.axis_index("s")
        nch = jnp.where(c == 0, _NCH0, _NCH1)
        cbase = jnp.where(c == 0, s * _NCH0, _NS * _NCH0 + s * _NCH1)
        pltpu.sync_copy(z_hbm, acc.at[pl.ds(s * _RPA, _RPA)])
        plsc.subcore_barrier()

        def gather(q, b):
            return pltpu.make_async_copy(
                hn_hbm.at[src_v.at[pl.ds(q * _CHUNK, _CHUNK)]],
                rows_v.at[b], gsem.at[b])

        def scat(q, b):
            return pltpu.make_async_copy(
                rows_v.at[b], acc.at[dst_v.at[q]], ssem.at[b])

        @pl.loop(0, nch // _WSTEP)
        def _(w):
            base = cbase + w * _WSTEP
            pltpu.sync_copy(
                src_hbm.at[pl.ds(base * _CHUNK, _WSTEP * _CHUNK)], src_v)
            pltpu.sync_copy(dst_hbm.at[pl.ds(base, _WSTEP)], dst_v)
            gather(0, 0).start()

            @pl.loop(0, _WSTEP, step=nbuf)
            def _(q0):
                for b in range(nbuf):
                    q = q0 + b
                    gather(q, b).wait()
                    scat(q, b).start(add=True)
                    bp = (b - 1) % nbuf

                    @pl.when(q >= 1)
                    def _():
                        scat(q - 1, bp).wait()

                    bn = (b + 1) % nbuf

                    @pl.when(q + 1 < _WSTEP)
                    def _():
                        gather(q + 1, bn).start()

            scat(_WSTEP - 1, (_WSTEP - 1) % nbuf).wait()

        plsc.subcore_barrier()
        pltpu.sync_copy(acc.at[pl.ds(s * _RPA, _RPA)],
                        out_hbm.at[pl.ds(c * _NPA + s * _RPA, _RPA)])

    return agg_kernel(hn, srcf, dstc, zrows)


_BLK = 1000
_NBLK = _N // _BLK


def _dot(a, b):
    return jnp.dot(a, b, precision=lax.Precision.HIGHEST,
                   preferred_element_type=_F32)


def _dot_t(a, b):
    return lax.dot_general(a, b, (((0,), (0,)), ((), ())),
                           precision=lax.Precision.HIGHEST,
                           preferred_element_type=_F32)


def _deg_rsq(dw_ref):
    deg = jnp.sum(dw_ref[...], axis=0) + 1.0
    return deg, lax.rsqrt(deg)


def _blk(deg=False):
    if deg:
        return pl.BlockSpec((_NW, _BLK, 1), lambda i: (0, i, 0))
    return pl.BlockSpec((_BLK, _D), lambda i: (i, 0))


def _full(shape):
    return pl.BlockSpec(shape, lambda i: tuple(0 for _ in shape))


def _mm_body(x_ref, w_ref, b_ref, o_ref):
    o_ref[...] = _dot(x_ref[...], w_ref[...]) + b_ref[...]


def _prep_body(h_ref, dw_ref, hn_ref, sf_ref):
    deg, rsq = _deg_rsq(dw_ref)
    h = h_ref[...]
    hn_ref[...] = h * rsq
    sf_ref[...] = h / deg


def _accum_body(p0_ref, p1_ref, dw_ref, sf_ref,
                agg_ref, s1_ref, s2_ref):
    _, rsq = _deg_rsq(dw_ref)
    agg = (p0_ref[...] + p1_ref[...]) * rsq + sf_ref[...]
    agg_ref[...] = agg

    @pl.when(pl.program_id(0) == 0)
    def _():
        s1_ref[...] = jnp.zeros_like(s1_ref)
        s2_ref[...] = jnp.zeros_like(s2_ref)

    s1_ref[...] += jnp.sum(agg, axis=0, keepdims=True)
    s2_ref[...] += jnp.sum(agg * agg, axis=0, keepdims=True)


def _bn_relu(agg, s1, s2, g, bt):
    m = s1 * (1.0 / _N)
    v = s2 * (1.0 / _N) - m * m
    return jnp.maximum((agg - m) * lax.rsqrt(v + 1e-5) * g + bt, 0.0)


def _layer2_body(agg_ref, s1_ref, s2_ref, g_ref, bt_ref, w_ref, b_ref,
                 dw_ref, hn_ref, sf_ref):
    deg, rsq = _deg_rsq(dw_ref)
    hr = _bn_relu(agg_ref[...], s1_ref[...], s2_ref[...], g_ref[...],
                  bt_ref[...])
    h2 = _dot(hr, w_ref[...]) + b_ref[...]
    hn_ref[...] = h2 * rsq
    sf_ref[...] = h2 / deg


def _pool_body(agg_ref, s1_ref, s2_ref, g_ref, bt_ref, batch_ref,
               ps_ref, cnt_ref):
    hr = _bn_relu(agg_ref[...], s1_ref[...], s2_ref[...], g_ref[...],
                  bt_ref[...])
    gi = lax.broadcasted_iota(jnp.int32, (_BLK, _G), 1)
    oh = (batch_ref[...] == gi).astype(_F32)

    @pl.when(pl.program_id(0) == 0)
    def _():
        ps_ref[...] = jnp.zeros_like(ps_ref)
        cnt_ref[...] = jnp.zeros_like(cnt_ref)

    ps_ref[...] += _dot_t(hr, oh)
    cnt_ref[...] += jnp.sum(oh, axis=0, keepdims=True)


def _head_body(ps_ref, cnt_ref, wl_ref, bl_ref, o_ref):
    pooled_t = ps_ref[...] / jnp.maximum(cnt_ref[...], 1.0)
    logits = _dot_t(pooled_t, wl_ref[...]) + bl_ref[...]
    mx = jnp.max(logits, axis=1, keepdims=True)
    lse = jnp.log(jnp.sum(jnp.exp(logits - mx), axis=1, keepdims=True)) + mx
    o_ref[...] = logits - lse


def _sds(shape):
    return jax.ShapeDtypeStruct(shape, _F32)


def _gcn_layer_tail(p, degw, sf):
    return pl.pallas_call(
        _accum_body,
        grid=(_NBLK,),
        in_specs=[_blk(), _blk(), _blk(True), _blk()],
        out_specs=[_blk(), _full((1, _D)), _full((1, _D))],
        out_shape=[_sds((_N, _D)), _sds((1, _D)), _sds((1, _D))],
    )(p[0], p[1], degw, sf)


def kernel(x, edge_index, batch, W1, b1, g1, bt1, W2, b2, g2, bt2, Wl, bl):
    src, dst = edge_index[0], edge_index[1]
    pad = _EPAD - _E
    srcf = jnp.concatenate([src, jnp.zeros((pad,), jnp.int32)])
    dump = _DUMP + jnp.arange(pad, dtype=jnp.int32) % (_NPA - _N)
    dstf = jnp.concatenate([dst, dump])
    dstc = dstf.reshape(_NW * _NCHUNK, _CHUNK)
    zdeg = jnp.zeros((_NPAD,), _F32)
    zrows = jnp.zeros((_RPA, _D), _F32)
    b1r, g1r, bt1r = b1.reshape(1, -1), g1.reshape(1, -1), bt1.reshape(1, -1)
    b2r, g2r, bt2r = b2.reshape(1, -1), g2.reshape(1, -1), bt2.reshape(1, -1)
    blr = bl.reshape(1, 2)
    batchc = batch.reshape(_N, 1)

    degw = _sc_degree(dstc, zdeg).reshape(_NW, _NPAD, 1)

    h1 = pl.pallas_call(
        _mm_body,
        grid=(_NBLK,),
        in_specs=[_blk(), _full((_D, _D)), _full((1, _D))],
        out_specs=_blk(),
        out_shape=_sds((_N, _D)),
    )(x, W1, b1r)

    hn1, sf1 = pl.pallas_call(
        _prep_body,
        grid=(_NBLK,),
        in_specs=[_blk(), _blk(True)],
        out_specs=[_blk(), _blk()],
        out_shape=[_sds((_N, _D)), _sds((_N, _D))],
    )(h1, degw)

    p1 = _sc_edge_agg(hn1, srcf, dstc, zrows).reshape(_NC, _NPA, _D)
    agg1, s1a, s2a = _gcn_layer_tail(p1, degw, sf1)

    hn2, sf2 = pl.pallas_call(
        _layer2_body,
        grid=(_NBLK,),
        in_specs=[_blk(), _full((1, _D)), _full((1, _D)), _full((1, _D)),
                  _full((1, _D)), _full((_D, _D)), _full((1, _D)),
                  _blk(True)],
        out_specs=[_blk(), _blk()],
        out_shape=[_sds((_N, _D)), _sds((_N, _D))],
    )(agg1, s1a, s2a, g1r, bt1r, W2, b2r, degw)

    p2 = _sc_edge_agg(hn2, srcf, dstc, zrows).reshape(_NC, _NPA, _D)
    agg2, s1b, s2b = _gcn_layer_tail(p2, degw, sf2)

    ps, cnt = pl.pallas_call(
        _pool_body,
        grid=(_NBLK,),
        in_specs=[_blk(), _full((1, _D)), _full((1, _D)), _full((1, _D)),
                  _full((1, _D)), pl.BlockSpec((_BLK, 1), lambda i: (i, 0))],
        out_specs=[_full((_D, _G)), _full((1, _G))],
        out_shape=[_sds((_D, _G)), _sds((1, _G))],
    )(agg2, s1b, s2b, g2r, bt2r, batchc)

    out = pl.pallas_call(_head_body, out_shape=_sds((_G, 2)))(
        ps, cnt, Wl, blr)
    return out


# speedup vs baseline: 1.4235x; 1.4235x over previous
import dataclasses
import functools

import jax
import jax.numpy as jnp
from jax import lax
from jax.experimental import pallas as pl
from jax.experimental.pallas import tpu as pltpu
from jax.experimental.pallas import tpu_sc as plsc

_N, _E, _D, _G = 10000, 320000, 128, 64
_NC, _NS = 2, 16
_NW = _NC * _NS
_CHUNK = 128
_NCHUNK = 80
_EPW = _CHUNK * _NCHUNK
_EPAD = _NW * _EPW
_NPAD = 10240
_RPS = _NPAD // _NS
_DUMP = _N
_NBUF = 2
_WSTEP = 8
_NCH0 = 152
_NCH1 = 8
_NPA = 10112
_RPA = _NPA // _NS

_F32 = jnp.float32


def _vmesh():
    return plsc.VectorSubcoreMesh(core_axis_name="c", subcore_axis_name="s")


def _sc_compiler_params():
    cp = pltpu.CompilerParams()
    if "needs_layout_passes" in pltpu.CompilerParams.__dataclass_fields__:
        cp = dataclasses.replace(cp, needs_layout_passes=False)
    return cp


def _sc_degree(dstc, zdeg):
    @functools.partial(
        pl.kernel,
        out_type=jax.ShapeDtypeStruct((_NW * _NPAD,), _F32),
        mesh=_vmesh(),
        scratch_types=[
            pltpu.VMEM((_NCHUNK, _CHUNK), jnp.int32),
            pltpu.VMEM((_NPAD,), _F32),
        ],
        compiler_params=_sc_compiler_params(),
    )
    def deg_kernel(dst_hbm, z_hbm, out_hbm, dst_v, local):
        c = lax.axis_index("c")
        s = lax.axis_index("s")
        wid = s * _NC + c
        pltpu.sync_copy(dst_hbm.at[pl.ds(wid * _NCHUNK, _NCHUNK)], dst_v)
        pltpu.sync_copy(z_hbm, local)
        ones = jnp.full((16,), 1.0, _F32)

        @pl.loop(0, _NCHUNK)
        def _(j):
            @pl.loop(0, _CHUNK // 16)
            def _(k):
                idx16 = dst_v[j, pl.ds(k * 16, 16)]
                plsc.addupdate_scatter(local, [idx16], ones)

        pltpu.sync_copy(local, out_hbm.at[pl.ds(wid * _NPAD, _NPAD)])

    return deg_kernel(dstc, zdeg)


def _sc_edge_agg(hn, srcf, dstc, zrows):
    nbuf = _NBUF

    @functools.partial(
        pl.kernel,
        out_type=jax.ShapeDtypeStruct((_NC * _NPA, _D), _F32),
        mesh=_vmesh(),
        scratch_types=[
            pltpu.VMEM((_WSTEP * _CHUNK,), jnp.int32),
            pltpu.VMEM((_WSTEP, _CHUNK), jnp.int32),
            pltpu.VMEM((nbuf, _CHUNK, _D), _F32),
            pltpu.VMEM_SHARED((_NPA, _D), _F32),
            pltpu.SemaphoreType.DMA((nbuf,)),
            pltpu.SemaphoreType.DMA((nbuf,)),
        ],
    )
    def agg_kernel(hn_hbm, src_hbm, dst_hbm, z_hbm, out_hbm,
                   src_v, dst_v, rows_v, acc, gsem, ssem):
        c = lax.axis_index("c")
        s = lax.axis_index("s")
        nch = jnp.where(c == 0, _NCH0, _NCH1)
        cbase = jnp.where(c == 0, s * _NCH0, _NS * _NCH0 + s * _NCH1)
        pltpu.sync_copy(z_hbm, acc.at[pl.ds(s * _RPA, _RPA)])
        plsc.subcore_barrier()

        def gather(q, b):
            return pltpu.make_async_copy(
                hn_hbm.at[src_v.at[pl.ds(q * _CHUNK, _CHUNK)]],
                rows_v.at[b], gsem.at[b])

        def scat(q, b):
            return pltpu.make_async_copy(
                rows_v.at[b], acc.at[dst_v.at[q]], ssem.at[b])

        @pl.loop(0, nch // _WSTEP)
        def _(w):
            base = cbase + w * _WSTEP
            pltpu.sync_copy(
                src_hbm.at[pl.ds(base * _CHUNK, _WSTEP * _CHUNK)], src_v)
            pltpu.sync_copy(dst_hbm.at[pl.ds(base, _WSTEP)], dst_v)
            gather(0, 0).start()

            @pl.loop(0, _WSTEP, step=nbuf)
            def _(q0):
                for b in range(nbuf):
                    q = q0 + b
                    gather(q, b).wait()
                    scat(q, b).start(add=True)
                    bp = (b - 1) % nbuf

                    @pl.when(q >= 1)
                    def _():
                        scat(q - 1, bp).wait()

                    bn = (b + 1) % nbuf

                    @pl.when(q + 1 < _WSTEP)
                    def _():
                        gather(q + 1, bn).start()

            scat(_WSTEP - 1, (_WSTEP - 1) % nbuf).wait()

        plsc.subcore_barrier()
        pltpu.sync_copy(acc.at[pl.ds(s * _RPA, _RPA)],
                        out_hbm.at[pl.ds(c * _NPA + s * _RPA, _RPA)])

    return agg_kernel(hn, srcf, dstc, zrows)


_BLK = 1000
_NBLK = _N // _BLK


def _dot(a, b):
    return jnp.dot(a, b, precision=lax.Precision.HIGHEST,
                   preferred_element_type=_F32)


def _dot_t(a, b):
    return lax.dot_general(a, b, (((0,), (0,)), ((), ())),
                           precision=lax.Precision.HIGHEST,
                           preferred_element_type=_F32)


def _deg_rsq(dw_ref):
    deg = jnp.sum(dw_ref[...], axis=0) + 1.0
    return deg, lax.rsqrt(deg)


def _blk(deg=False):
    if deg:
        return pl.BlockSpec((_NW, _BLK, 1), lambda i: (0, i, 0))
    return pl.BlockSpec((_BLK, _D), lambda i: (i, 0))


def _full(shape):
    return pl.BlockSpec(shape, lambda i: tuple(0 for _ in shape))


def _mm_body(x_ref, w_ref, b_ref, o_ref):
    o_ref[...] = _dot(x_ref[...], w_ref[...]) + b_ref[...]


def _prep_body(h_ref, dw_ref, hn_ref, sf_ref):
    deg, rsq = _deg_rsq(dw_ref)
    h = h_ref[...]
    hn_ref[...] = h * rsq
    sf_ref[...] = h / deg


def _accum_body(p0_ref, p1_ref, dw_ref, sf_ref,
                agg_ref, s1_ref, s2_ref):
    _, rsq = _deg_rsq(dw_ref)
    agg = (p0_ref[...] + p1_ref[...]) * rsq + sf_ref[...]
    agg_ref[...] = agg

    @pl.when(pl.program_id(0) == 0)
    def _():
        s1_ref[...] = jnp.zeros_like(s1_ref)
        s2_ref[...] = jnp.zeros_like(s2_ref)

    s1_ref[...] += jnp.sum(agg, axis=0, keepdims=True)
    s2_ref[...] += jnp.sum(agg * agg, axis=0, keepdims=True)


def _bn_relu(agg, s1, s2, g, bt):
    m = s1 * (1.0 / _N)
    v = s2 * (1.0 / _N) - m * m
    return jnp.maximum((agg - m) * lax.rsqrt(v + 1e-5) * g + bt, 0.0)


def _layer2_body(agg_ref, s1_ref, s2_ref, g_ref, bt_ref, w_ref, b_ref,
                 dw_ref, hn_ref, sf_ref):
    deg, rsq = _deg_rsq(dw_ref)
    hr = _bn_relu(agg_ref[...], s1_ref[...], s2_ref[...], g_ref[...],
                  bt_ref[...])
    h2 = _dot(hr, w_ref[...]) + b_ref[...]
    hn_ref[...] = h2 * rsq
    sf_ref[...] = h2 / deg


def _pool_body(agg_ref, s1_ref, s2_ref, g_ref, bt_ref, batch_ref,
               ps_ref, cnt_ref):
    hr = _bn_relu(agg_ref[...], s1_ref[...], s2_ref[...], g_ref[...],
                  bt_ref[...])
    gi = lax.broadcasted_iota(jnp.int32, (_BLK, _G), 1)
    oh = (batch_ref[...] == gi).astype(_F32)

    @pl.when(pl.program_id(0) == 0)
    def _():
        ps_ref[...] = jnp.zeros_like(ps_ref)
        cnt_ref[...] = jnp.zeros_like(cnt_ref)

    ps_ref[...] += _dot_t(hr, oh)
    cnt_ref[...] += jnp.sum(oh, axis=0, keepdims=True)


def _head_body(ps_ref, cnt_ref, wl_ref, bl_ref, o_ref):
    pooled_t = ps_ref[...] / jnp.maximum(cnt_ref[...], 1.0)
    logits = _dot_t(pooled_t, wl_ref[...]) + bl_ref[...]
    mx = jnp.max(logits, axis=1, keepdims=True)
    lse = jnp.log(jnp.sum(jnp.exp(logits - mx), axis=1, keepdims=True)) + mx
    o_ref[...] = logits - lse


def _sds(shape):
    return jax.ShapeDtypeStruct(shape, _F32)


def _gcn_layer_tail(p, degw, sf):
    return pl.pallas_call(
        _accum_body,
        grid=(_NBLK,),
        in_specs=[_blk(), _blk(), _blk(True), _blk()],
        out_specs=[_blk(), _full((1, _D)), _full((1, _D))],
        out_shape=[_sds((_N, _D)), _sds((1, _D)), _sds((1, _D))],
    )(p[0], p[1], degw, sf)


def kernel(x, edge_index, batch, W1, b1, g1, bt1, W2, b2, g2, bt2, Wl, bl):
    src, dst = edge_index[0], edge_index[1]
    pad = _EPAD - _E
    srcf = jnp.concatenate([src, jnp.zeros((pad,), jnp.int32)])
    dump = _DUMP + jnp.arange(pad, dtype=jnp.int32) % (_NPA - _N)
    dstf = jnp.concatenate([dst, dump])
    dstc = dstf.reshape(_NW * _NCHUNK, _CHUNK)
    zdeg = jnp.zeros((_NPAD,), _F32)
    zrows = jnp.zeros((_RPA, _D), _F32)
    b1r, g1r, bt1r = b1.reshape(1, -1), g1.reshape(1, -1), bt1.reshape(1, -1)
    b2r, g2r, bt2r = b2.reshape(1, -1), g2.reshape(1, -1), bt2.reshape(1, -1)
    blr = bl.reshape(1, 2)
    batchc = batch.reshape(_N, 1)

    degw = _sc_degree(dstc, zdeg).reshape(_NW, _NPAD, 1)

    h1 = pl.pallas_call(
        _mm_body,
        grid=(_NBLK,),
        in_specs=[_blk(), _full((_D, _D)), _full((1, _D))],
        out_specs=_blk(),
        out_shape=_sds((_N, _D)),
    )(x, W1, b1r)

    hn1, sf1 = pl.pallas_call(
        _prep_body,
        grid=(_NBLK,),
        in_specs=[_blk(), _blk(True)],
        out_specs=[_blk(), _blk()],
        out_shape=[_sds((_N, _D)), _sds((_N, _D))],
    )(h1, degw)

    p1 = _sc_edge_agg(hn1, srcf, dstc, zrows).reshape(_NC, _NPA, _D)
    agg1, s1a, s2a = _gcn_layer_tail(p1, degw, sf1)

    hn2, sf2 = pl.pallas_call(
        _layer2_body,
        grid=(_NBLK,),
        in_specs=[_blk(), _full((1, _D)), _full((1, _D)), _full((1, _D)),
                  _full((1, _D)), _full((_D, _D)), _full((1, _D)),
                  _blk(True)],
        out_specs=[_blk(), _blk()],
        out_shape=[_sds((_N, _D)), _sds((_N, _D))],
    )(agg1, s1a, s2a, g1r, bt1r, W2, b2r, degw)

    p2 = _sc_edge_agg(hn2, srcf, dstc, zrows).reshape(_NC, _NPA, _D)
    agg2, s1b, s2b = _gcn_layer_tail(p2, degw, sf2)

    ps, cnt = pl.pallas_call(
        _pool_body,
        grid=(_NBLK,),
        in_specs=[_blk(), _full((1, _D)), _full((1, _D)), _full((1, _D)),
                  _full((1, _D)), pl.BlockSpec((_BLK, 1), lambda i: (i, 0))],
        out_specs=[_full((_D, _G)), _full((1, _G))],
        out_shape=[_sds((_D, _G)), _sds((1, _G))],
    )(agg2, s1b, s2b, g2r, bt2r, batchc)

    out = pl.pallas_call(_head_body, out_shape=_sds((_G, 2)))(
        ps, cnt, Wl, blr)
    return out


# speedup vs baseline: 1.4475x; 1.0169x over previous
import dataclasses
import functools

import jax
import jax.numpy as jnp
from jax import lax
from jax.experimental import pallas as pl
from jax.experimental.pallas import tpu as pltpu
from jax.experimental.pallas import tpu_sc as plsc

_N, _E, _D, _G = 10000, 320000, 128, 64
_NC, _NS = 2, 16
_NW = _NC * _NS
_CHUNK = 128
_NCHUNK = 80
_EPW = _CHUNK * _NCHUNK
_EPAD = _NW * _EPW
_NPAD = 10240
_RPS = _NPAD // _NS
_DUMP = _N
_NBUF = 2
_WSTEP = 8
_NCH0 = 144
_NCH1 = 16
_NPA = 10112
_RPA = _NPA // _NS

_F32 = jnp.float32


def _vmesh():
    return plsc.VectorSubcoreMesh(core_axis_name="c", subcore_axis_name="s")


def _sc_compiler_params():
    cp = pltpu.CompilerParams()
    if "needs_layout_passes" in pltpu.CompilerParams.__dataclass_fields__:
        cp = dataclasses.replace(cp, needs_layout_passes=False)
    return cp


def _sc_degree(dstc, zdeg):
    @functools.partial(
        pl.kernel,
        out_type=jax.ShapeDtypeStruct((_NW * _NPAD,), _F32),
        mesh=_vmesh(),
        scratch_types=[
            pltpu.VMEM((_NCHUNK, _CHUNK), jnp.int32),
            pltpu.VMEM((_NPAD,), _F32),
        ],
        compiler_params=_sc_compiler_params(),
    )
    def deg_kernel(dst_hbm, z_hbm, out_hbm, dst_v, local):
        c = lax.axis_index("c")
        s = lax.axis_index("s")
        wid = s * _NC + c
        pltpu.sync_copy(dst_hbm.at[pl.ds(wid * _NCHUNK, _NCHUNK)], dst_v)
        pltpu.sync_copy(z_hbm, local)
        ones = jnp.full((16,), 1.0, _F32)

        @pl.loop(0, _NCHUNK)
        def _(j):
            @pl.loop(0, _CHUNK // 16)
            def _(k):
                idx16 = dst_v[j, pl.ds(k * 16, 16)]
                plsc.addupdate_scatter(local, [idx16], ones)

        pltpu.sync_copy(local, out_hbm.at[pl.ds(wid * _NPAD, _NPAD)])

    return deg_kernel(dstc, zdeg)


def _sc_edge_agg(hn, srcf, dstc, zrows):
    nbuf = _NBUF

    @functools.partial(
        pl.kernel,
        out_type=jax.ShapeDtypeStruct((_NC * _NPA, _D), _F32),
        mesh=_vmesh(),
        scratch_types=[
            pltpu.VMEM((_WSTEP * _CHUNK,), jnp.int32),
            pltpu.VMEM((_WSTEP, _CHUNK), jnp.int32),
            pltpu.VMEM((nbuf, _CHUNK, _D), _F32),
            pltpu.VMEM_SHARED((_NPA, _D), _F32),
            pltpu.SemaphoreType.DMA((nbuf,)),
            pltpu.SemaphoreType.DMA((nbuf,)),
        ],
    )
    def agg_kernel(hn_hbm, src_hbm, dst_hbm, z_hbm, out_hbm,
                   src_v, dst_v, rows_v, acc, gsem, ssem):
        c = lax.axis_index("c")
        s = lax.axis_index("s")
        nch = jnp.where(c == 0, _NCH0, _NCH1)
        cbase = jnp.where(c == 0, s * _NCH0, _NS * _NCH0 + s * _NCH1)
        pltpu.sync_copy(z_hbm, acc.at[pl.ds(s * _RPA, _RPA)])
        plsc.subcore_barrier()

        def gather(q, b):
            return pltpu.make_async_copy(
                hn_hbm.at[src_v.at[pl.ds(q * _CHUNK, _CHUNK)]],
                rows_v.at[b], gsem.at[b])

        def scat(q, b):
            return pltpu.make_async_copy(
                rows_v.at[b], acc.at[dst_v.at[q]], ssem.at[b])

        @pl.loop(0, nch // _WSTEP)
        def _(w):
            base = cbase + w * _WSTEP
            pltpu.sync_copy(
                src_hbm.at[pl.ds(base * _CHUNK, _WSTEP * _CHUNK)], src_v)
            pltpu.sync_copy(dst_hbm.at[pl.ds(base, _WSTEP)], dst_v)
            gather(0, 0).start()

            @pl.loop(0, _WSTEP, step=nbuf)
            def _(q0):
                for b in range(nbuf):
                    q = q0 + b
                    gather(q, b).wait()
                    scat(q, b).start(add=True)
                    bp = (b - 1) % nbuf

                    @pl.when(q >= 1)
                    def _():
                        scat(q - 1, bp).wait()

                    bn = (b + 1) % nbuf

                    @pl.when(q + 1 < _WSTEP)
                    def _():
                        gather(q + 1, bn).start()

            scat(_WSTEP - 1, (_WSTEP - 1) % nbuf).wait()

        plsc.subcore_barrier()
        pltpu.sync_copy(acc.at[pl.ds(s * _RPA, _RPA)],
                        out_hbm.at[pl.ds(c * _NPA + s * _RPA, _RPA)])

    return agg_kernel(hn, srcf, dstc, zrows)


_BLK = 1000
_NBLK = _N // _BLK


def _dot(a, b):
    return jnp.dot(a, b, precision=lax.Precision.HIGHEST,
                   preferred_element_type=_F32)


def _dot_t(a, b):
    return lax.dot_general(a, b, (((0,), (0,)), ((), ())),
                           precision=lax.Precision.HIGHEST,
                           preferred_element_type=_F32)


def _deg_rsq(dw_ref):
    deg = jnp.sum(dw_ref[...], axis=0) + 1.0
    return deg, lax.rsqrt(deg)


def _blk(deg=False):
    if deg:
        return pl.BlockSpec((_NW, _BLK, 1), lambda i: (0, i, 0))
    return pl.BlockSpec((_BLK, _D), lambda i: (i, 0))


def _full(shape):
    return pl.BlockSpec(shape, lambda i: tuple(0 for _ in shape))


def _mm_body(x_ref, w_ref, b_ref, o_ref):
    o_ref[...] = _dot(x_ref[...], w_ref[...]) + b_ref[...]


def _prep_body(h_ref, dw_ref, hn_ref, sf_ref):
    deg, rsq = _deg_rsq(dw_ref)
    h = h_ref[...]
    hn_ref[...] = h * rsq
    sf_ref[...] = h / deg


def _accum_body(p0_ref, p1_ref, dw_ref, sf_ref,
                agg_ref, s1_ref, s2_ref):
    _, rsq = _deg_rsq(dw_ref)
    agg = (p0_ref[...] + p1_ref[...]) * rsq + sf_ref[...]
    agg_ref[...] = agg

    @pl.when(pl.program_id(0) == 0)
    def _():
        s1_ref[...] = jnp.zeros_like(s1_ref)
        s2_ref[...] = jnp.zeros_like(s2_ref)

    s1_ref[...] += jnp.sum(agg, axis=0, keepdims=True)
    s2_ref[...] += jnp.sum(agg * agg, axis=0, keepdims=True)


def _bn_relu(agg, s1, s2, g, bt):
    m = s1 * (1.0 / _N)
    v = s2 * (1.0 / _N) - m * m
    return jnp.maximum((agg - m) * lax.rsqrt(v + 1e-5) * g + bt, 0.0)


def _layer2_body(agg_ref, s1_ref, s2_ref, g_ref, bt_ref, w_ref, b_ref,
                 dw_ref, hn_ref, sf_ref):
    deg, rsq = _deg_rsq(dw_ref)
    hr = _bn_relu(agg_ref[...], s1_ref[...], s2_ref[...], g_ref[...],
                  bt_ref[...])
    h2 = _dot(hr, w_ref[...]) + b_ref[...]
    hn_ref[...] = h2 * rsq
    sf_ref[...] = h2 / deg


def _pool_body(agg_ref, s1_ref, s2_ref, g_ref, bt_ref, batch_ref,
               ps_ref, cnt_ref):
    hr = _bn_relu(agg_ref[...], s1_ref[...], s2_ref[...], g_ref[...],
                  bt_ref[...])
    gi = lax.broadcasted_iota(jnp.int32, (_BLK, _G), 1)
    oh = (batch_ref[...] == gi).astype(_F32)

    @pl.when(pl.program_id(0) == 0)
    def _():
        ps_ref[...] = jnp.zeros_like(ps_ref)
        cnt_ref[...] = jnp.zeros_like(cnt_ref)

    ps_ref[...] += _dot_t(hr, oh)
    cnt_ref[...] += jnp.sum(oh, axis=0, keepdims=True)


def _head_body(ps_ref, cnt_ref, wl_ref, bl_ref, o_ref):
    pooled_t = ps_ref[...] / jnp.maximum(cnt_ref[...], 1.0)
    logits = _dot_t(pooled_t, wl_ref[...]) + bl_ref[...]
    mx = jnp.max(logits, axis=1, keepdims=True)
    lse = jnp.log(jnp.sum(jnp.exp(logits - mx), axis=1, keepdims=True)) + mx
    o_ref[...] = logits - lse


def _sds(shape):
    return jax.ShapeDtypeStruct(shape, _F32)


def _gcn_layer_tail(p, degw, sf):
    return pl.pallas_call(
        _accum_body,
        grid=(_NBLK,),
        in_specs=[_blk(), _blk(), _blk(True), _blk()],
        out_specs=[_blk(), _full((1, _D)), _full((1, _D))],
        out_shape=[_sds((_N, _D)), _sds((1, _D)), _sds((1, _D))],
    )(p[0], p[1], degw, sf)


def kernel(x, edge_index, batch, W1, b1, g1, bt1, W2, b2, g2, bt2, Wl, bl):
    src, dst = edge_index[0], edge_index[1]
    pad = _EPAD - _E
    srcf = jnp.concatenate([src, jnp.zeros((pad,), jnp.int32)])
    dump = _DUMP + jnp.arange(pad, dtype=jnp.int32) % (_NPA - _N)
    dstf = jnp.concatenate([dst, dump])
    dstc = dstf.reshape(_NW * _NCHUNK, _CHUNK)
    zdeg = jnp.zeros((_NPAD,), _F32)
    zrows = jnp.zeros((_RPA, _D), _F32)
    b1r, g1r, bt1r = b1.reshape(1, -1), g1.reshape(1, -1), bt1.reshape(1, -1)
    b2r, g2r, bt2r = b2.reshape(1, -1), g2.reshape(1, -1), bt2.reshape(1, -1)
    blr = bl.reshape(1, 2)
    batchc = batch.reshape(_N, 1)

    degw = _sc_degree(dstc, zdeg).reshape(_NW, _NPAD, 1)

    h1 = pl.pallas_call(
        _mm_body,
        grid=(_NBLK,),
        in_specs=[_blk(), _full((_D, _D)), _full((1, _D))],
        out_specs=_blk(),
        out_shape=_sds((_N, _D)),
    )(x, W1, b1r)

    hn1, sf1 = pl.pallas_call(
        _prep_body,
        grid=(_NBLK,),
        in_specs=[_blk(), _blk(True)],
        out_specs=[_blk(), _blk()],
        out_shape=[_sds((_N, _D)), _sds((_N, _D))],
    )(h1, degw)

    p1 = _sc_edge_agg(hn1, srcf, dstc, zrows).reshape(_NC, _NPA, _D)
    agg1, s1a, s2a = _gcn_layer_tail(p1, degw, sf1)

    hn2, sf2 = pl.pallas_call(
        _layer2_body,
        grid=(_NBLK,),
        in_specs=[_blk(), _full((1, _D)), _full((1, _D)), _full((1, _D)),
                  _full((1, _D)), _full((_D, _D)), _full((1, _D)),
                  _blk(True)],
        out_specs=[_blk(), _blk()],
        out_shape=[_sds((_N, _D)), _sds((_N, _D))],
    )(agg1, s1a, s2a, g1r, bt1r, W2, b2r, degw)

    p2 = _sc_edge_agg(hn2, srcf, dstc, zrows).reshape(_NC, _NPA, _D)
    agg2, s1b, s2b = _gcn_layer_tail(p2, degw, sf2)

    ps, cnt = pl.pallas_call(
        _pool_body,
        grid=(_NBLK,),
        in_specs=[_blk(), _full((1, _D)), _full((1, _D)), _full((1, _D)),
                  _full((1, _D)), pl.BlockSpec((_BLK, 1), lambda i: (i, 0))],
        out_specs=[_full((_D, _G)), _full((1, _G))],
        out_shape=[_sds((_D, _G)), _sds((1, _G))],
    )(agg2, s1b, s2b, g2r, bt2r, batchc)

    out = pl.pallas_call(_head_body, out_shape=_sds((_G, 2)))(
        ps, cnt, Wl, blr)
    return out
